# Initial kernel scaffold; baseline (speedup 1.0000x reference)
import jax
import jax.numpy as jnp
from jax.experimental import pallas as pl


def kernel(data, phy_edge_index, net_edge_index, mul_edge_index, mul_emb, phy_emb, net_emb, Ws, bs, Wp, bp, Wn, bn, Wo, bo, Wpo, bpo, Wno, bno, g1, be1, g2, be2, g3, be3):
    raise NotImplementedError("write your pallas kernel here")


# speedup vs baseline: 6.1594x; 6.1594x over previous
import functools

import jax
import jax.numpy as jnp
from jax import lax
from jax.experimental import pallas as pl
from jax.experimental.pallas import tpu as pltpu
from jax.experimental.pallas import tpu_sc as plsc

_TOPK = 20
_NW = 32
_CPW = 320
_S = 16
_BR = 256
_BN1 = 400


def _topk_call(emb_pad, n_real, interpret=False):
    NP = emb_pad.shape[0]

    def body(full_ref, blk_ref, idx_ref, sc_ref, acc_ref):
        k = pl.program_id(1)
        neg = jnp.float32(-jnp.inf)

        @pl.when(k == 0)
        def _init():
            w = full_ref[...]
            nrmf = jnp.maximum(jnp.sqrt(jnp.sum(w * w, axis=1)), 1e-30)
            wb = blk_ref[...]
            nrmb = jnp.maximum(
                jnp.sqrt(jnp.sum(wb * wb, axis=1, keepdims=True)), 1e-30)
            dot = lax.dot_general(wb, w, (((1,), (1,)), ((), ())),
                                  preferred_element_type=jnp.float32)
            scores = dot / (nrmb * nrmf[None, :])
            icol = lax.broadcasted_iota(jnp.int32, scores.shape, 1)
            sc_ref[...] = jnp.where(icol < n_real, scores, neg)

        scores = sc_ref[...]
        col = lax.broadcasted_iota(jnp.int32, scores.shape, 1)
        m = jnp.max(scores, axis=1, keepdims=True)
        am = jnp.min(jnp.where(scores == m, col, NP), axis=1).astype(jnp.int32)
        am = jnp.minimum(am, n_real - 1)
        col20 = lax.broadcasted_iota(jnp.int32, (_BR, _TOPK), 1)
        acc_ref[...] = jnp.where(col20 == k, am[:, None], acc_ref[...])
        sc_ref[...] = jnp.where(col == am[:, None], neg, scores)

        @pl.when(k == _TOPK - 1)
        def _flush():
            idx_ref[...] = acc_ref[...]

    return pl.pallas_call(
        body,
        grid=(NP // _BR, _TOPK),
        in_specs=[
            pl.BlockSpec((NP, 64), lambda i, k: (0, 0)),
            pl.BlockSpec((_BR, 64), lambda i, k: (i, 0)),
        ],
        out_specs=pl.BlockSpec((_BR, _TOPK), lambda i, k: (i, 0)),
        out_shape=jax.ShapeDtypeStruct((NP, _TOPK), jnp.int32),
        scratch_shapes=[
            pltpu.VMEM((_BR, NP), jnp.float32),
            pltpu.VMEM((_BR, _TOPK), jnp.int32),
        ],
        interpret=interpret,
    )(emb_pad, emb_pad)


def _sc_gather_mean(table, idx_flat, D):
    NP = _NW * _CPW
    CH = D // 16
    K = _TOPK
    mesh = plsc.VectorSubcoreMesh(core_axis_name="c", subcore_axis_name="s")

    @functools.partial(
        pl.kernel,
        out_type=jax.ShapeDtypeStruct((NP, D), jnp.float32),
        scratch_types=[
            pltpu.VMEM((_S * K,), jnp.int32),
            pltpu.VMEM((_S * K, D), jnp.float32),
            pltpu.VMEM((_S, D), jnp.float32),
            pltpu.SemaphoreType.DMA,
        ],
        mesh=mesh,
        compiler_params=pltpu.CompilerParams(use_tc_tiling_on_sc=False),
    )
    def k(table_hbm, idx_hbm, out_hbm, idx_v, rows_v, out_v, sem):
        wid = lax.axis_index("s") * 2 + lax.axis_index("c")
        base = wid * _CPW

        def step(s, carry):
            node0 = base + s * _S
            pltpu.sync_copy(idx_hbm.at[pl.ds(node0 * K, _S * K)], idx_v)
            handles = []
            for j in range(_S * K // 64):
                handles.append(pltpu.async_copy(
                    table_hbm.at[idx_v.at[pl.ds(j * 64, 64)]],
                    rows_v.at[pl.ds(j * 64, 64)], sem))
            for h in handles:
                h.wait()

            def node_body(n, c2):
                def red(r, accs):
                    return tuple(accs[c] + rows_v[n * K + r, pl.ds(c * 16, 16)]
                                 for c in range(CH))
                accs = lax.fori_loop(
                    0, K, red,
                    tuple(jnp.zeros((16,), jnp.float32) for _ in range(CH)))
                for c in range(CH):
                    out_v[n, pl.ds(c * 16, 16)] = accs[c] * (1.0 / K)
                return c2
            lax.fori_loop(0, _S, node_body, 0)
            pltpu.sync_copy(out_v, out_hbm.at[pl.ds(node0, _S)])
            return carry
        lax.fori_loop(0, _CPW // _S, step, 0)

    return k(table, idx_flat)


def _stage1_call(xm_t, mul_emb, Ws, bs, g1s, be1, Wo, bo, interpret=False):
    N = xm_t.shape[0]
    B = 4

    def body(xm_ref, me_ref, Ws_ref, bs_ref, g1_ref, be1_ref, Wo_ref, bo_ref,
             out_ref, op_ref, on_ref):
        xm = xm_ref[...]
        me = me_ref[...]
        Wsv = Ws_ref[...]
        bsv = bs_ref[...]
        g1v = g1_ref[...]
        be1v = be1_ref[...]
        Wov = Wo_ref[...]
        bov = bo_ref[...]
        outs = []
        for b in range(B):
            xb = xm[:, b * 60:(b + 1) * 60]
            g = jnp.maximum(jnp.dot(xb, Wsv, preferred_element_type=jnp.float32)
                            + bsv, 0.0)
            o = g * me
            o = jnp.maximum(o * g1v + be1v, 0.0)
            outs.append(jnp.dot(o, Wov, preferred_element_type=jnp.float32)
                        + bov)
        out_ref[...] = jnp.stack(outs, axis=0)
        op_ref[...] = jnp.concatenate([ob[:, :16] for ob in outs], axis=1)
        on_ref[...] = jnp.concatenate([ob[:, 16:] for ob in outs], axis=1)

    grid = N // _BN1
    full = lambda shape: pl.BlockSpec(shape, lambda i: tuple(0 for _ in shape))
    return pl.pallas_call(
        body,
        grid=(grid,),
        in_specs=[
            pl.BlockSpec((_BN1, 240), lambda i: (i, 0)),
            pl.BlockSpec((_BN1, 64), lambda i: (i, 0)),
            full((60, 64)), full((1, 64)), full((1, 64)), full((1, 64)),
            full((64, 64)), full((1, 64)),
        ],
        out_specs=[
            pl.BlockSpec((B, _BN1, 64), lambda i: (0, i, 0)),
            pl.BlockSpec((_BN1, 64), lambda i: (i, 0)),
            pl.BlockSpec((_BN1, 192), lambda i: (i, 0)),
        ],
        out_shape=[
            jax.ShapeDtypeStruct((B, N, 64), jnp.float32),
            jax.ShapeDtypeStruct((N, 64), jnp.float32),
            jax.ShapeDtypeStruct((N, 192), jnp.float32),
        ],
        interpret=interpret,
    )(xm_t, mul_emb, Ws, bs, g1s, be1, Wo, bo)


def _stage2_call(opa, ona, phy_emb, net_emb, Wp, bp, Wn, bnn, g2s, be2,
                 g3s, be3, Wpo, bpo, Wno, bno, interpret=False):
    N = opa.shape[0]
    B = 4

    def body(opa_ref, ona_ref, pe_ref, ne_ref, Wp_ref, bp_ref, Wn_ref, bn_ref,
             g2_ref, be2_ref, g3_ref, be3_ref, Wpo_ref, bpo_ref, Wno_ref,
             bno_ref, res_ref):
        opav = opa_ref[...]
        onav = ona_ref[...]
        pe = pe_ref[...]
        ne = ne_ref[...]
        res = []
        for b in range(B):
            t = jnp.maximum(
                jnp.dot(opav[:, b * 16:(b + 1) * 16], Wp_ref[...],
                        preferred_element_type=jnp.float32) + bp_ref[...], 0.0)
            t = t * pe
            t = jnp.maximum(t * g2_ref[...] + be2_ref[...], 0.0)
            rp = jnp.dot(t, Wpo_ref[...], preferred_element_type=jnp.float32) \
                + bpo_ref[...]
            u = jnp.maximum(
                jnp.dot(onav[:, b * 48:(b + 1) * 48], Wn_ref[...],
                        preferred_element_type=jnp.float32) + bn_ref[...], 0.0)
            u = u * ne
            u = jnp.maximum(u * g3_ref[...] + be3_ref[...], 0.0)
            rn = jnp.dot(u, Wno_ref[...], preferred_element_type=jnp.float32) \
                + bno_ref[...]
            res.append(jnp.concatenate([rp, rn], axis=1))
        res_ref[...] = jnp.stack(res, axis=0)

    grid = N // _BN1
    full = lambda shape: pl.BlockSpec(shape, lambda i: tuple(0 for _ in shape))
    return pl.pallas_call(
        body,
        grid=(grid,),
        in_specs=[
            pl.BlockSpec((_BN1, 64), lambda i: (i, 0)),
            pl.BlockSpec((_BN1, 192), lambda i: (i, 0)),
            pl.BlockSpec((_BN1, 64), lambda i: (i, 0)),
            pl.BlockSpec((_BN1, 64), lambda i: (i, 0)),
            full((16, 64)), full((1, 64)), full((48, 64)), full((1, 64)),
            full((1, 64)), full((1, 64)), full((1, 64)), full((1, 64)),
            full((64, 1)), full((1, 1)), full((64, 3)), full((1, 3)),
        ],
        out_specs=pl.BlockSpec((B, _BN1, 4), lambda i: (0, i, 0)),
        out_shape=jax.ShapeDtypeStruct((B, N, 4), jnp.float32),
        interpret=interpret,
    )(opa, ona, phy_emb, net_emb, Wp, bp, Wn, bnn, g2s, be2, g3s, be3,
      Wpo, bpo, Wno, bno)


def kernel(data, phy_edge_index, net_edge_index, mul_edge_index, mul_emb,
           phy_emb, net_emb, Ws, bs, Wp, bp, Wn, bn, Wo, bo, Wpo, bpo,
           Wno, bno, g1, be1, g2, be2, g3, be3):
    B, N, F = data.shape
    NP = _NW * _CPW
    bnscale = jnp.float32(1.0 / jnp.sqrt(jnp.float32(1.0 + 1e-5)))

    def pad_rows(a):
        return jnp.pad(a, ((0, NP - N), (0, 0)))

    r2 = lambda v: v.reshape(1, -1)

    idx_m = _topk_call(pad_rows(mul_emb), N)
    idx_p = _topk_call(pad_rows(phy_emb), N)
    idx_n = _topk_call(pad_rows(net_emb), N)

    x_t = jnp.transpose(data, (1, 0, 2)).reshape(N, B * F)
    xm_t = _sc_gather_mean(x_t, idx_m.reshape(-1), B * F)[:N]
    out1, op_t, on_t = _stage1_call(
        xm_t, mul_emb, Ws, r2(bs), r2(g1 * bnscale), r2(be1), Wo, r2(bo))

    opa = _sc_gather_mean(op_t, idx_p.reshape(-1), 64)[:N]
    ona = _sc_gather_mean(on_t, idx_n.reshape(-1), 192)[:N]
    res = _stage2_call(
        opa, ona, phy_emb, net_emb, Wp, r2(bp), Wn, r2(bn),
        r2(g2 * bnscale), r2(be2), r2(g3 * bnscale), r2(be3),
        Wpo, bpo.reshape(1, 1), Wno, r2(bno))

    out = out1.reshape(B * N, 64)
    phy_out = res[:, :, 0:1].reshape(-1, 1)
    net_out = res[:, :, 1:4].reshape(-1, 3)
    return (out, phy_out, net_out)
